# baseline (speedup 1.0000x reference)
import functools

import jax
import jax.numpy as jnp
from jax import lax
from jax.experimental import pallas as pl
from jax.experimental.pallas import tpu as pltpu
from jax.experimental.pallas import tpu_sc as plsc

N = 10000
D = 128
C = 16
E = 320000

NC = 2
NS = 16
NW = NC * NS

BLK = 128
BPT_AGG = -(-(-(-E // (NW * BLK))) // 8) * 8
E_PAD = NW * BPT_AGG * BLK
EROWS = E_PAD // BLK
BPT_DEG = EROWS // NS

N_PAD = 10240
NPT = N_PAD // NS


def _deg_body(src_hbm, dst_hbm, out_hbm, sidx_v, didx_v, deg_v):
    c = lax.axis_index("c")
    s = lax.axis_index("s")
    zeros = jnp.zeros((16,), jnp.float32)
    ones = jnp.full((16,), 1.0, jnp.float32)
    for k in range(2 * N_PAD // 16):
        deg_v[pl.ds(k * 16, 16)] = zeros
    pltpu.sync_copy(src_hbm.at[pl.ds(s * BPT_DEG, BPT_DEG)], sidx_v)
    pltpu.sync_copy(dst_hbm.at[pl.ds(s * BPT_DEG, BPT_DEG)], didx_v)

    def step(b, carry):
        for j in range(BLK // 16):
            sv = sidx_v[b, pl.ds(j * 16, 16)]
            plsc.addupdate_scatter(deg_v, [sv], ones)
            dv = didx_v[b, pl.ds(j * 16, 16)]
            plsc.addupdate_scatter(deg_v, [dv + N_PAD], ones)
        return carry

    lax.fori_loop(0, BPT_DEG, step, 0)
    pltpu.sync_copy(deg_v.at[pl.ds(c * N_PAD, N_PAD)], out_hbm.at[c, s])


_deg_call = functools.partial(
    pl.kernel,
    out_type=jax.ShapeDtypeStruct((NC, NS, N_PAD), jnp.float32),
    mesh=plsc.VectorSubcoreMesh(core_axis_name="c", subcore_axis_name="s"),
    scratch_types=[
        pltpu.VMEM((BPT_DEG, BLK), jnp.int32),
        pltpu.VMEM((BPT_DEG, BLK), jnp.int32),
        pltpu.VMEM((2 * N_PAD,), jnp.float32),
    ],
    compiler_params=pltpu.CompilerParams(needs_layout_passes=False),
)(_deg_body)


K_RING = 2
HB = 16
B0 = 32
B1 = 128
NPH = -(-max(B0, B1) // HB)
NGH = HB // K_RING


def _agg_body(h_hbm, src_hbm, dst_hbm, out_hbm, sidx_v, didx_v, rows_v,
              zero_v, agg_sh, gsem):
    c = lax.axis_index("c")
    s = lax.axis_index("s")
    for i in range(16):
        for j in range(D // 16):
            zero_v[i, pl.ds(j * 16, 16)] = jnp.zeros((16,), jnp.float32)
    for k in range(NPT // 16):
        pltpu.sync_copy(zero_v, agg_sh.at[pl.ds(s * NPT + k * 16, 16)])
    plsc.subcore_barrier()

    nblk = jnp.where(c == 0, B0, B1)
    base = jnp.where(c == 0, s * B0, NS * B0 + s * B1)
    for ph in range(NPH):
        r0 = jnp.minimum(base + ph * HB, EROWS - HB)
        ng = jnp.clip((nblk - ph * HB) // K_RING, 0, NGH)
        pltpu.sync_copy(src_hbm.at[pl.ds(r0, HB)], sidx_v)
        pltpu.sync_copy(dst_hbm.at[pl.ds(r0, HB)], didx_v)

        @pl.when(ng > 0)
        def _():
            for k in range(K_RING):
                pltpu.async_copy(h_hbm.at[sidx_v.at[k]], rows_v.at[k],
                                 gsem.at[k])

        def group(g, carry):
            bb = g * K_RING
            for k in range(K_RING):
                i = bb + k
                pltpu.make_async_copy(h_hbm.at[sidx_v.at[i]], rows_v.at[k],
                                      gsem.at[k]).wait()
                pltpu.sync_copy(rows_v.at[k], agg_sh.at[didx_v.at[i]],
                                add=True)

                @pl.when(g < ng - 1)
                def _():
                    pltpu.async_copy(h_hbm.at[sidx_v.at[i + K_RING]],
                                     rows_v.at[k], gsem.at[k])
            return carry

        lax.fori_loop(0, ng, group, 0)
    plsc.subcore_barrier()
    pltpu.sync_copy(agg_sh.at[pl.ds(s * NPT, NPT)],
                    out_hbm.at[c, pl.ds(s * NPT, NPT)])


_agg_call = functools.partial(
    pl.kernel,
    out_type=jax.ShapeDtypeStruct((NC, N_PAD, D), jnp.float32),
    mesh=plsc.VectorSubcoreMesh(core_axis_name="c", subcore_axis_name="s"),
    scratch_types=[
        pltpu.VMEM((HB, BLK), jnp.int32),
        pltpu.VMEM((HB, BLK), jnp.int32),
        pltpu.VMEM((K_RING, BLK, D), jnp.float32),
        pltpu.VMEM((16, D), jnp.float32),
        pltpu.VMEM_SHARED((N_PAD, D), jnp.float32),
        pltpu.SemaphoreType.DMA((K_RING,)),
    ],
)(_agg_body)


def _norm(deg_col):
    return jnp.where(deg_col > 0.0,
                     lax.rsqrt(jnp.maximum(deg_col, 1.0)), 0.0)


def _col_sum(page):
    ones = jnp.ones((NS, 1), jnp.float32)
    return lax.dot_general(page, ones, (((0,), (0,)), ((), ())),
                           preferred_element_type=jnp.float32)


def _scale_body(x_ref, deg_ref, o_ref, nd_ref):
    od = _col_sum(deg_ref[0])
    idg = _col_sum(deg_ref[1])
    o_ref[...] = x_ref[...] * _norm(od)
    nd_ref[...] = _norm(idg)


def _dense_body(agg_ref, nd_ref, w1_ref, b1_ref, w2_ref, b2_ref, o_ref):
    t = (agg_ref[0] + agg_ref[1]) * nd_ref[...]
    y = jnp.dot(t, w1_ref[...], preferred_element_type=jnp.float32)
    y = jnp.maximum(y + b1_ref[...], 0.0)
    rows = lax.broadcasted_iota(jnp.int32, (N_PAD, 1), 0)
    y = jnp.where(rows < N, y, 0.0)
    m = jnp.sum(y, axis=0, keepdims=True) * (1.0 / N)
    o_ref[...] = jnp.dot(m, w2_ref[...], preferred_element_type=jnp.float32) \
        + b2_ref[...]


def kernel(x, edge_index, W1, b1, W2, b2):
    src = edge_index[0]
    dst = edge_index[1]
    pad = E_PAD - E
    padv = jnp.full((pad,), N, jnp.int32)
    src_p = jnp.concatenate([src, padv]).reshape(EROWS, BLK)
    dst_p = jnp.concatenate([dst, padv]).reshape(EROWS, BLK)
    x_p = jnp.zeros((N_PAD, D), jnp.float32).at[:N].set(x)

    deg = _deg_call(src_p, dst_p)

    h, norm_dst = pl.pallas_call(
        _scale_body,
        out_shape=(jax.ShapeDtypeStruct((N_PAD, D), jnp.float32),
                   jax.ShapeDtypeStruct((N_PAD, 1), jnp.float32)),
    )(x_p, deg)

    agg = _agg_call(h, src_p, dst_p)

    out = pl.pallas_call(
        _dense_body,
        out_shape=jax.ShapeDtypeStruct((1, C), jnp.float32),
    )(agg, norm_dst, W1, b1.reshape(1, D), W2, b2.reshape(1, C))
    return out


# speedup vs baseline: 1.0673x; 1.0673x over previous
import functools

import jax
import jax.numpy as jnp
from jax import lax
from jax.experimental import pallas as pl
from jax.experimental.pallas import tpu as pltpu
from jax.experimental.pallas import tpu_sc as plsc

N = 10000
D = 128
C = 16
E = 320000

NC = 2
NS = 16
NW = NC * NS

BLK = 128
BPT_AGG = -(-(-(-E // (NW * BLK))) // 8) * 8
E_PAD = NW * BPT_AGG * BLK
EROWS = E_PAD // BLK
BPT_DEG = EROWS // NS

N_PAD = 10240
NPT = N_PAD // NS


def _deg_body(src_hbm, dst_hbm, out_hbm, sidx_v, didx_v, deg_v):
    c = lax.axis_index("c")
    s = lax.axis_index("s")
    zeros = jnp.zeros((16,), jnp.float32)
    ones = jnp.full((16,), 1.0, jnp.float32)
    for k in range(2 * N_PAD // 16):
        deg_v[pl.ds(k * 16, 16)] = zeros
    pltpu.sync_copy(src_hbm.at[pl.ds(s * BPT_DEG, BPT_DEG)], sidx_v)
    pltpu.sync_copy(dst_hbm.at[pl.ds(s * BPT_DEG, BPT_DEG)], didx_v)

    def step(b, carry):
        for j in range(BLK // 16):
            sv = sidx_v[b, pl.ds(j * 16, 16)]
            plsc.addupdate_scatter(deg_v, [sv], ones)
            dv = didx_v[b, pl.ds(j * 16, 16)]
            plsc.addupdate_scatter(deg_v, [dv + N_PAD], ones)
        return carry

    lax.fori_loop(0, BPT_DEG, step, 0)
    pltpu.sync_copy(deg_v.at[pl.ds(c * N_PAD, N_PAD)], out_hbm.at[c, s])


_deg_call = functools.partial(
    pl.kernel,
    out_type=jax.ShapeDtypeStruct((NC, NS, N_PAD), jnp.float32),
    mesh=plsc.VectorSubcoreMesh(core_axis_name="c", subcore_axis_name="s"),
    scratch_types=[
        pltpu.VMEM((BPT_DEG, BLK), jnp.int32),
        pltpu.VMEM((BPT_DEG, BLK), jnp.int32),
        pltpu.VMEM((2 * N_PAD,), jnp.float32),
    ],
    compiler_params=pltpu.CompilerParams(needs_layout_passes=False),
)(_deg_body)


K_RING = 2
HB = 16
B0 = 80
B1 = 80
NPH = -(-max(B0, B1) // HB)
NGH = HB // K_RING


def _agg_body(h_hbm, src_hbm, dst_hbm, out_hbm, sidx_v, didx_v, rows_v,
              zero_v, agg_sh, gsem):
    c = lax.axis_index("c")
    s = lax.axis_index("s")
    for i in range(16):
        for j in range(D // 16):
            zero_v[i, pl.ds(j * 16, 16)] = jnp.zeros((16,), jnp.float32)
    for k in range(NPT // 16):
        pltpu.sync_copy(zero_v, agg_sh.at[pl.ds(s * NPT + k * 16, 16)])
    plsc.subcore_barrier()

    nblk = jnp.where(c == 0, B0, B1)
    base = jnp.where(c == 0, s * B0, NS * B0 + s * B1)
    for ph in range(NPH):
        r0 = jnp.minimum(base + ph * HB, EROWS - HB)
        ng = jnp.clip((nblk - ph * HB) // K_RING, 0, NGH)
        pltpu.sync_copy(src_hbm.at[pl.ds(r0, HB)], sidx_v)
        pltpu.sync_copy(dst_hbm.at[pl.ds(r0, HB)], didx_v)

        @pl.when(ng > 0)
        def _():
            for k in range(K_RING):
                pltpu.async_copy(h_hbm.at[sidx_v.at[k]], rows_v.at[k],
                                 gsem.at[k])

        def group(g, carry):
            bb = g * K_RING
            for k in range(K_RING):
                i = bb + k
                pltpu.make_async_copy(h_hbm.at[sidx_v.at[i]], rows_v.at[k],
                                      gsem.at[k]).wait()

                @pl.when(g < ng - 1)
                def _():
                    pltpu.async_copy(h_hbm.at[sidx_v.at[i + K_RING]],
                                     rows_v.at[k], gsem.at[k])
            return carry

        lax.fori_loop(0, ng, group, 0)
    plsc.subcore_barrier()
    pltpu.sync_copy(agg_sh.at[pl.ds(s * NPT, NPT)],
                    out_hbm.at[c, pl.ds(s * NPT, NPT)])


_agg_call = functools.partial(
    pl.kernel,
    out_type=jax.ShapeDtypeStruct((NC, N_PAD, D), jnp.float32),
    mesh=plsc.VectorSubcoreMesh(core_axis_name="c", subcore_axis_name="s"),
    scratch_types=[
        pltpu.VMEM((HB, BLK), jnp.int32),
        pltpu.VMEM((HB, BLK), jnp.int32),
        pltpu.VMEM((K_RING, BLK, D), jnp.float32),
        pltpu.VMEM((16, D), jnp.float32),
        pltpu.VMEM_SHARED((N_PAD, D), jnp.float32),
        pltpu.SemaphoreType.DMA((K_RING,)),
    ],
)(_agg_body)


def _norm(deg_col):
    return jnp.where(deg_col > 0.0,
                     lax.rsqrt(jnp.maximum(deg_col, 1.0)), 0.0)


def _col_sum(page):
    ones = jnp.ones((NS, 1), jnp.float32)
    return lax.dot_general(page, ones, (((0,), (0,)), ((), ())),
                           preferred_element_type=jnp.float32)


def _scale_body(x_ref, deg_ref, o_ref, nd_ref):
    od = _col_sum(deg_ref[0])
    idg = _col_sum(deg_ref[1])
    o_ref[...] = x_ref[...] * _norm(od)
    nd_ref[...] = _norm(idg)


def _dense_body(agg_ref, nd_ref, w1_ref, b1_ref, w2_ref, b2_ref, o_ref):
    t = (agg_ref[0] + agg_ref[1]) * nd_ref[...]
    y = jnp.dot(t, w1_ref[...], preferred_element_type=jnp.float32)
    y = jnp.maximum(y + b1_ref[...], 0.0)
    rows = lax.broadcasted_iota(jnp.int32, (N_PAD, 1), 0)
    y = jnp.where(rows < N, y, 0.0)
    m = jnp.sum(y, axis=0, keepdims=True) * (1.0 / N)
    o_ref[...] = jnp.dot(m, w2_ref[...], preferred_element_type=jnp.float32) \
        + b2_ref[...]


def kernel(x, edge_index, W1, b1, W2, b2):
    src = edge_index[0]
    dst = edge_index[1]
    pad = E_PAD - E
    padv = jnp.full((pad,), N, jnp.int32)
    src_p = jnp.concatenate([src, padv]).reshape(EROWS, BLK)
    dst_p = jnp.concatenate([dst, padv]).reshape(EROWS, BLK)
    x_p = jnp.zeros((N_PAD, D), jnp.float32).at[:N].set(x)

    deg = _deg_call(src_p, dst_p)

    h, norm_dst = pl.pallas_call(
        _scale_body,
        out_shape=(jax.ShapeDtypeStruct((N_PAD, D), jnp.float32),
                   jax.ShapeDtypeStruct((N_PAD, 1), jnp.float32)),
    )(x_p, deg)

    agg = _agg_call(h, src_p, dst_p)

    out = pl.pallas_call(
        _dense_body,
        out_shape=jax.ShapeDtypeStruct((1, C), jnp.float32),
    )(agg, norm_dst, W1, b1.reshape(1, D), W2, b2.reshape(1, C))
    return out


# speedup vs baseline: 1.3096x; 1.2270x over previous
import functools

import jax
import jax.numpy as jnp
from jax import lax
from jax.experimental import pallas as pl
from jax.experimental.pallas import tpu as pltpu
from jax.experimental.pallas import tpu_sc as plsc

N = 10000
D = 128
C = 16
E = 320000

NC = 2
NS = 16
NW = NC * NS

BLK = 128
BPT_AGG = -(-(-(-E // (NW * BLK))) // 8) * 8
E_PAD = NW * BPT_AGG * BLK
EROWS = E_PAD // BLK
BPT_DEG = EROWS // NS

N_PAD = 10240
NPT = N_PAD // NS


def _deg_body(edges_hbm, out_hbm, idx_v, deg_v):
    c = lax.axis_index("c")
    s = lax.axis_index("s")
    zeros = jnp.zeros((16,), jnp.float32)
    ones = jnp.full((16,), 1.0, jnp.float32)
    for k in range(N_PAD // 16):
        deg_v[pl.ds(k * 16, 16)] = zeros
    pltpu.sync_copy(edges_hbm.at[c, pl.ds(s * BPT_DEG, BPT_DEG)], idx_v)

    def step(b, carry):
        for j in range(BLK // 16):
            ev = idx_v[b, pl.ds(j * 16, 16)]
            plsc.addupdate_scatter(deg_v, [ev], ones)
        return carry

    lax.fori_loop(0, BPT_DEG, step, 0)
    pltpu.sync_copy(deg_v, out_hbm.at[c, s])


_deg_call = functools.partial(
    pl.kernel,
    out_type=jax.ShapeDtypeStruct((NC, NS, N_PAD), jnp.float32),
    mesh=plsc.VectorSubcoreMesh(core_axis_name="c", subcore_axis_name="s"),
    scratch_types=[
        pltpu.VMEM((BPT_DEG, BLK), jnp.int32),
        pltpu.VMEM((N_PAD,), jnp.float32),
    ],
    compiler_params=pltpu.CompilerParams(needs_layout_passes=False),
)(_deg_body)


K_RING = 2
HB = 16
B0 = 80
B1 = 80
NPH = -(-max(B0, B1) // HB)
NGH = HB // K_RING


def _agg_body(h_hbm, src_hbm, dst_hbm, out_hbm, sidx_v, didx_v, rows_v,
              zero_v, agg_sh, gsem):
    c = lax.axis_index("c")
    s = lax.axis_index("s")
    for i in range(16):
        for j in range(D // 16):
            zero_v[i, pl.ds(j * 16, 16)] = jnp.zeros((16,), jnp.float32)
    for k in range(NPT // 16):
        pltpu.sync_copy(zero_v, agg_sh.at[pl.ds(s * NPT + k * 16, 16)])
    plsc.subcore_barrier()

    nblk = jnp.where(c == 0, B0, B1)
    base = jnp.where(c == 0, s * B0, NS * B0 + s * B1)
    for ph in range(NPH):
        r0 = jnp.minimum(base + ph * HB, EROWS - HB)
        ng = jnp.clip((nblk - ph * HB) // K_RING, 0, NGH)
        pltpu.sync_copy(src_hbm.at[pl.ds(r0, HB)], sidx_v)
        pltpu.sync_copy(dst_hbm.at[pl.ds(r0, HB)], didx_v)

        @pl.when(ng > 0)
        def _():
            for k in range(K_RING):
                pltpu.async_copy(h_hbm.at[sidx_v.at[k]], rows_v.at[k],
                                 gsem.at[k])

        def group(g, carry):
            bb = g * K_RING
            for k in range(K_RING):
                i = bb + k
                pltpu.make_async_copy(h_hbm.at[sidx_v.at[i]], rows_v.at[k],
                                      gsem.at[k]).wait()
                pltpu.sync_copy(rows_v.at[k], agg_sh.at[didx_v.at[i]],
                                add=True)

                @pl.when(g < ng - 1)
                def _():
                    pltpu.async_copy(h_hbm.at[sidx_v.at[i + K_RING]],
                                     rows_v.at[k], gsem.at[k])
            return carry

        lax.fori_loop(0, ng, group, 0)
    plsc.subcore_barrier()
    pltpu.sync_copy(agg_sh.at[pl.ds(s * NPT, NPT)],
                    out_hbm.at[c, pl.ds(s * NPT, NPT)])


_agg_call = functools.partial(
    pl.kernel,
    out_type=jax.ShapeDtypeStruct((NC, N_PAD, D), jnp.float32),
    mesh=plsc.VectorSubcoreMesh(core_axis_name="c", subcore_axis_name="s"),
    scratch_types=[
        pltpu.VMEM((HB, BLK), jnp.int32),
        pltpu.VMEM((HB, BLK), jnp.int32),
        pltpu.VMEM((K_RING, BLK, D), jnp.float32),
        pltpu.VMEM((16, D), jnp.float32),
        pltpu.VMEM_SHARED((N_PAD, D), jnp.float32),
        pltpu.SemaphoreType.DMA((K_RING,)),
    ],
)(_agg_body)


def _norm(deg_col):
    return jnp.where(deg_col > 0.0,
                     lax.rsqrt(jnp.maximum(deg_col, 1.0)), 0.0)


def _col_sum(page):
    ones = jnp.ones((NS, 1), jnp.float32)
    return lax.dot_general(page, ones, (((0,), (0,)), ((), ())),
                           preferred_element_type=jnp.float32)


def _scale_body(x_ref, deg_ref, o_ref, nd_ref):
    od = _col_sum(deg_ref[0])
    idg = _col_sum(deg_ref[1])
    o_ref[...] = x_ref[...] * _norm(od)
    nd_ref[...] = _norm(idg)


def _dense_body(agg_ref, nd_ref, w1_ref, b1_ref, w2_ref, b2_ref, o_ref):
    t = (agg_ref[0] + agg_ref[1]) * nd_ref[...]
    y = jnp.dot(t, w1_ref[...], preferred_element_type=jnp.float32)
    y = jnp.maximum(y + b1_ref[...], 0.0)
    rows = lax.broadcasted_iota(jnp.int32, (N_PAD, 1), 0)
    y = jnp.where(rows < N, y, 0.0)
    m = jnp.sum(y, axis=0, keepdims=True) * (1.0 / N)
    o_ref[...] = jnp.dot(m, w2_ref[...], preferred_element_type=jnp.float32) \
        + b2_ref[...]


def kernel(x, edge_index, W1, b1, W2, b2):
    src = edge_index[0]
    dst = edge_index[1]
    pad = E_PAD - E
    padv = jnp.full((pad,), N, jnp.int32)
    src_p = jnp.concatenate([src, padv]).reshape(EROWS, BLK)
    dst_p = jnp.concatenate([dst, padv]).reshape(EROWS, BLK)
    x_p = jnp.zeros((N_PAD, D), jnp.float32).at[:N].set(x)

    deg = _deg_call(jnp.stack([src_p, dst_p]))

    h, norm_dst = pl.pallas_call(
        _scale_body,
        out_shape=(jax.ShapeDtypeStruct((N_PAD, D), jnp.float32),
                   jax.ShapeDtypeStruct((N_PAD, 1), jnp.float32)),
    )(x_p, deg)

    agg = _agg_call(h, src_p, dst_p)

    out = pl.pallas_call(
        _dense_body,
        out_shape=jax.ShapeDtypeStruct((1, C), jnp.float32),
    )(agg, norm_dst, W1, b1.reshape(1, D), W2, b2.reshape(1, C))
    return out
